# baseline (speedup 1.0000x reference)
import jax
import jax.numpy as jnp
from jax.experimental import pallas as pl
from jax.experimental.pallas import tpu as pltpu

_KW = 4


def _fused_kernel(low_ref, high_ref,
                  w1l_ref, b1l_ref, w2l_ref, b2l_ref,
                  w1h_ref, b1h_ref, w2h_ref, b2h_ref,
                  flo_hbm, fhi_hbm, flo_s, fhi_s, sems):
    n = pl.program_id(0)
    low = low_ref[0]
    high = high_ref[0]
    hw = low.shape[1]
    inv_hw = 1.0 / hw

    gap_low = jnp.sum(low, axis=1, keepdims=True) * inv_hw
    gap_high = jnp.sum(high, axis=1, keepdims=True) * inv_hw

    def branch(g, w1, b1, w2, b2):
        h = jax.lax.dot_general(w1[...], g, (((1,), (0,)), ((), ())),
                                preferred_element_type=jnp.float32) + b1[...]
        h = jnp.maximum(h, 0.0)
        return jax.lax.dot_general(w2[...], h, (((1,), (0,)), ((), ())),
                                   preferred_element_type=jnp.float32) + b2[...]

    low_vec = branch(gap_low, w1l_ref, b1l_ref, w2l_ref, b2l_ref)
    high_vec = branch(gap_high, w1h_ref, b1h_ref, w2h_ref, b2h_ref)

    def soft1(v):
        m = jnp.max(v, axis=0, keepdims=True)
        e = jnp.exp(v - m)
        return e / jnp.sum(e, axis=0, keepdims=True) + 1.0

    flo_s[...] = low * soft1(low_vec) + low_vec
    fhi_s[...] = high * soft1(high_vec)

    chunk = hw // _KW
    for k in range(_KW):
        ds = pl.ds(k * chunk, chunk)
        pltpu.make_async_copy(flo_s.at[:, ds], flo_hbm.at[n, :, ds],
                              sems.at[0, k]).start()
        pltpu.make_async_copy(fhi_s.at[:, ds], fhi_hbm.at[n, :, ds],
                              sems.at[1, k]).start()
    for k in range(_KW):
        ds = pl.ds(k * chunk, chunk)
        pltpu.make_async_copy(flo_s.at[:, ds], flo_hbm.at[n, :, ds],
                              sems.at[0, k]).wait()
        pltpu.make_async_copy(fhi_s.at[:, ds], fhi_hbm.at[n, :, ds],
                              sems.at[1, k]).wait()


def _bn_fold(gamma, beta, mean, var, eps=1e-5):
    s = gamma / jnp.sqrt(var + eps)
    return s, beta - mean * s


def kernel(low, high, fc_low_w, fc_low_b, bn_low_1_gamma, bn_low_1_beta,
           bn_low_1_mean, bn_low_1_var, fcs0_w, fcs0_b, bn_low_2_gamma,
           bn_low_2_beta, bn_low_2_mean, bn_low_2_var, fc_high_w, fc_high_b,
           bn_high_1_gamma, bn_high_1_beta, bn_high_1_mean, bn_high_1_var,
           fcs1_w, fcs1_b, bn_high_2_gamma, bn_high_2_beta, bn_high_2_mean,
           bn_high_2_var):
    N, C, H, W = low.shape
    HW = H * W
    low_f = low.reshape(N, C, HW)
    high_f = high.reshape(N, C, HW)

    def fold(w1, b1, bn1, w2, b2, bn2):
        s1, t1 = _bn_fold(*bn1)
        s2, t2 = _bn_fold(*bn2)
        return (w1 * s1[:, None], (b1 * s1 + t1)[:, None],
                w2 * s2[:, None], (b2 * s2 + t2)[:, None])

    w1l, b1l, w2l, b2l = fold(
        fc_low_w, fc_low_b,
        (bn_low_1_gamma, bn_low_1_beta, bn_low_1_mean, bn_low_1_var),
        fcs0_w, fcs0_b,
        (bn_low_2_gamma, bn_low_2_beta, bn_low_2_mean, bn_low_2_var))
    w1h, b1h, w2h, b2h = fold(
        fc_high_w, fc_high_b,
        (bn_high_1_gamma, bn_high_1_beta, bn_high_1_mean, bn_high_1_var),
        fcs1_w, fcs1_b,
        (bn_high_2_gamma, bn_high_2_beta, bn_high_2_mean, bn_high_2_var))

    cr = w1l.shape[0]
    full = lambda shape: pl.BlockSpec(shape, lambda i: (0,) * len(shape))
    slab = pl.BlockSpec((1, C, HW), lambda i: (i, 0, 0))

    flo, fhi = pl.pallas_call(
        _fused_kernel,
        out_shape=(jax.ShapeDtypeStruct((N, C, HW), low.dtype),
                   jax.ShapeDtypeStruct((N, C, HW), high.dtype)),
        grid=(N,),
        in_specs=[slab, slab,
                  full((cr, C)), full((cr, 1)), full((C, cr)), full((C, 1)),
                  full((cr, C)), full((cr, 1)), full((C, cr)), full((C, 1))],
        out_specs=(pl.BlockSpec(memory_space=pl.ANY),
                   pl.BlockSpec(memory_space=pl.ANY)),
        scratch_shapes=[pltpu.VMEM((C, HW), jnp.float32),
                        pltpu.VMEM((C, HW), jnp.float32),
                        pltpu.SemaphoreType.DMA((2, _KW))],
        compiler_params=pltpu.CompilerParams(
            dimension_semantics=("parallel",)),
    )(low_f, high_f, w1l, b1l, w2l, b2l, w1h, b1h, w2h, b2h)

    return flo.reshape(N, C, H, W), fhi.reshape(N, C, H, W)


# speedup vs baseline: 1.0103x; 1.0103x over previous
import jax
import jax.numpy as jnp
from jax.experimental import pallas as pl
from jax.experimental.pallas import tpu as pltpu

_KW = 4


def _fused_kernel(low_ref, high_ref,
                  w1l_ref, b1l_ref, w2l_ref, b2l_ref,
                  w1h_ref, b1h_ref, w2h_ref, b2h_ref,
                  flo_hbm, fhi_hbm, flo_s, fhi_s, sems):
    n = pl.program_id(0)
    nsteps = pl.num_programs(0)
    slot = jax.lax.rem(n, 2)
    low = low_ref[0]
    high = high_ref[0]
    hw = low.shape[1]
    inv_hw = 1.0 / hw

    chunk = hw // _KW

    def start_writes(s, step):
        for k in range(_KW):
            ds = pl.ds(k * chunk, chunk)
            pltpu.make_async_copy(flo_s.at[s, :, ds], flo_hbm.at[step, :, ds],
                                  sems.at[s, 0, k]).start()
            pltpu.make_async_copy(fhi_s.at[s, :, ds], fhi_hbm.at[step, :, ds],
                                  sems.at[s, 1, k]).start()

    def wait_writes(s, step):
        for k in range(_KW):
            ds = pl.ds(k * chunk, chunk)
            pltpu.make_async_copy(flo_s.at[s, :, ds], flo_hbm.at[step, :, ds],
                                  sems.at[s, 0, k]).wait()
            pltpu.make_async_copy(fhi_s.at[s, :, ds], fhi_hbm.at[step, :, ds],
                                  sems.at[s, 1, k]).wait()

    @pl.when(n >= 2)
    def _():
        wait_writes(slot, n - 2)

    gap_low = jnp.sum(low, axis=1, keepdims=True) * inv_hw
    gap_high = jnp.sum(high, axis=1, keepdims=True) * inv_hw

    def branch(g, w1, b1, w2, b2):
        h = jax.lax.dot_general(w1[...], g, (((1,), (0,)), ((), ())),
                                preferred_element_type=jnp.float32) + b1[...]
        h = jnp.maximum(h, 0.0)
        return jax.lax.dot_general(w2[...], h, (((1,), (0,)), ((), ())),
                                   preferred_element_type=jnp.float32) + b2[...]

    low_vec = branch(gap_low, w1l_ref, b1l_ref, w2l_ref, b2l_ref)
    high_vec = branch(gap_high, w1h_ref, b1h_ref, w2h_ref, b2h_ref)

    def soft1(v):
        m = jnp.max(v, axis=0, keepdims=True)
        e = jnp.exp(v - m)
        return e / jnp.sum(e, axis=0, keepdims=True) + 1.0

    flo_s[slot] = low * soft1(low_vec) + low_vec
    fhi_s[slot] = high * soft1(high_vec)

    start_writes(slot, n)

    @pl.when(n == nsteps - 1)
    def _():
        @pl.when(nsteps >= 2)
        def _():
            wait_writes(jax.lax.rem(n + 1, 2), n - 1)
        wait_writes(slot, n)


def _bn_fold(gamma, beta, mean, var, eps=1e-5):
    s = gamma / jnp.sqrt(var + eps)
    return s, beta - mean * s


def kernel(low, high, fc_low_w, fc_low_b, bn_low_1_gamma, bn_low_1_beta,
           bn_low_1_mean, bn_low_1_var, fcs0_w, fcs0_b, bn_low_2_gamma,
           bn_low_2_beta, bn_low_2_mean, bn_low_2_var, fc_high_w, fc_high_b,
           bn_high_1_gamma, bn_high_1_beta, bn_high_1_mean, bn_high_1_var,
           fcs1_w, fcs1_b, bn_high_2_gamma, bn_high_2_beta, bn_high_2_mean,
           bn_high_2_var):
    N, C, H, W = low.shape
    HW = H * W
    low_f = low.reshape(N, C, HW)
    high_f = high.reshape(N, C, HW)

    def fold(w1, b1, bn1, w2, b2, bn2):
        s1, t1 = _bn_fold(*bn1)
        s2, t2 = _bn_fold(*bn2)
        return (w1 * s1[:, None], (b1 * s1 + t1)[:, None],
                w2 * s2[:, None], (b2 * s2 + t2)[:, None])

    w1l, b1l, w2l, b2l = fold(
        fc_low_w, fc_low_b,
        (bn_low_1_gamma, bn_low_1_beta, bn_low_1_mean, bn_low_1_var),
        fcs0_w, fcs0_b,
        (bn_low_2_gamma, bn_low_2_beta, bn_low_2_mean, bn_low_2_var))
    w1h, b1h, w2h, b2h = fold(
        fc_high_w, fc_high_b,
        (bn_high_1_gamma, bn_high_1_beta, bn_high_1_mean, bn_high_1_var),
        fcs1_w, fcs1_b,
        (bn_high_2_gamma, bn_high_2_beta, bn_high_2_mean, bn_high_2_var))

    cr = w1l.shape[0]
    full = lambda shape: pl.BlockSpec(shape, lambda i: (0,) * len(shape))
    slab = pl.BlockSpec((1, C, HW), lambda i: (i, 0, 0))

    flo, fhi = pl.pallas_call(
        _fused_kernel,
        out_shape=(jax.ShapeDtypeStruct((N, C, HW), low.dtype),
                   jax.ShapeDtypeStruct((N, C, HW), high.dtype)),
        grid=(N,),
        in_specs=[slab, slab,
                  full((cr, C)), full((cr, 1)), full((C, cr)), full((C, 1)),
                  full((cr, C)), full((cr, 1)), full((C, cr)), full((C, 1))],
        out_specs=(pl.BlockSpec(memory_space=pl.ANY),
                   pl.BlockSpec(memory_space=pl.ANY)),
        scratch_shapes=[pltpu.VMEM((2, C, HW), jnp.float32),
                        pltpu.VMEM((2, C, HW), jnp.float32),
                        pltpu.SemaphoreType.DMA((2, 2, _KW))],
        compiler_params=pltpu.CompilerParams(
            dimension_semantics=("parallel",)),
    )(low_f, high_f, w1l, b1l, w2l, b2l, w1h, b1h, w2h, b2h)

    return flo.reshape(N, C, H, W), fhi.reshape(N, C, H, W)
